# baseline (speedup 1.0000x reference)
import jax
import jax.numpy as jnp
from jax import lax
from jax.experimental import pallas as pl
from jax.experimental.pallas import tpu as pltpu
from jax.experimental.pallas import tpu_sc as plsc

D = 128
NC = 2
NS = 16
NW = NC * NS


def _mlp_u_body(vfeat_ref, w1_ref, b1_ref, wve_ref, bve_ref, vrw_ref, u_ref):
    t = jnp.dot(vfeat_ref[...], w1_ref[...],
                preferred_element_type=jnp.float32) + b1_ref[...]
    u = jnp.dot(t, wve_ref[...],
                preferred_element_type=jnp.float32) + bve_ref[...]
    u_ref[...] = vrw_ref[...] * u


def _edge_body(p_ref, ers_ref, erw_ref, wev_ref, bev_ref, fe_ref, v_ref):
    fe = (p_ref[0] + p_ref[1]) / ers_ref[...]
    fe_ref[...] = fe
    v = jnp.dot(fe, wev_ref[...],
                preferred_element_type=jnp.float32) + bev_ref[...]
    v_ref[...] = erw_ref[...] * v


def _vout_body(p_ref, vrs_ref, out_ref):
    out_ref[...] = (p_ref[0] + p_ref[1]) / vrs_ref[...]


_ZC = 80
_CHUNK = 128
_G = 20
_NCHUNK = 80
_NG = _NCHUNK // _G


def _sc_pass(table, comb_idx, acc_rows):
    zeros = jnp.zeros((_ZC, D), jnp.float32)
    zr = acc_rows // NS
    kz = zr // _ZC

    mesh = plsc.VectorSubcoreMesh(core_axis_name="c", subcore_axis_name="s")

    def body(table_hbm, comb_hbm, zeros_hbm, out_hbm,
             acc, idx_v, rows_v, sem0, sem1, isem):
        c = lax.axis_index("c")
        s = lax.axis_index("s")
        wid = c * NS + s
        pltpu.sync_copy(comb_hbm.at[wid, pl.ds(0, _G)], idx_v.at[0])
        pltpu.sync_copy(zeros_hbm, rows_v.at[0, pl.ds(0, _ZC)])
        for k in range(kz):
            pltpu.sync_copy(rows_v.at[0, pl.ds(0, _ZC)],
                            acc.at[pl.ds(s * zr + k * _ZC, _ZC)])
        plsc.subcore_barrier()

        pltpu.async_copy(comb_hbm.at[wid, pl.ds(_G, _G)], idx_v.at[1], isem)

        def gth(slot, k, b, sem):
            return pltpu.make_async_copy(
                table_hbm.at[idx_v.at[slot, k, 0]], rows_v.at[b], sem)

        sems = (sem0, sem1)
        gth(0, 0, 0, sem0).start()

        def group_body(g, carry):
            slot = lax.rem(g, 2)
            nslot = lax.rem(g + 1, 2)
            for k in range(_G):
                b = k % 2
                gth(slot, k, b, sems[b]).wait()
                if k < _G - 1:
                    gth(slot, k + 1, 1 - b, sems[1 - b]).start()
                else:
                    @pl.when(g < _NG - 1)
                    def _():
                        pltpu.make_async_copy(
                            comb_hbm.at[wid, pl.ds((g + 1) * _G, _G)],
                            idx_v.at[nslot], isem).wait()
                        gth(nslot, 0, 1 - b, sems[1 - b]).start()
                pltpu.sync_copy(rows_v.at[b], acc.at[idx_v.at[slot, k, 1]],
                                add=True)

            @pl.when(g < _NG - 2)
            def _():
                pltpu.async_copy(comb_hbm.at[wid, pl.ds((g + 2) * _G, _G)],
                                 idx_v.at[slot], isem)
            return carry

        lax.fori_loop(0, _NG, group_body, 0)
        plsc.subcore_barrier()
        pltpu.sync_copy(acc.at[pl.ds(s * zr, zr)],
                        out_hbm.at[c, pl.ds(s * zr, zr)])

    return pl.kernel(
        body,
        out_type=jax.ShapeDtypeStruct((NC, acc_rows, D), jnp.float32),
        mesh=mesh,
        scratch_types=[
            pltpu.VMEM_SHARED((acc_rows, D), jnp.float32),
            pltpu.VMEM((2, _G, 2, _CHUNK), jnp.int32),
            pltpu.VMEM((2, _CHUNK, D), jnp.float32),
            pltpu.SemaphoreType.DMA,
            pltpu.SemaphoreType.DMA,
            pltpu.SemaphoreType.DMA,
        ],
    )(table, comb_idx, zeros)


def _pack_idx(gather_idx, scatter_idx, pad_scatter):
    e = gather_idx.shape[0]
    epad = NW * _NCHUNK * _CHUNK
    g = jnp.concatenate(
        [gather_idx, jnp.zeros((epad - e,), jnp.int32)]).reshape(
            NW, _NCHUNK, 1, _CHUNK)
    s = jnp.concatenate(
        [scatter_idx, jnp.full((epad - e,), pad_scatter, jnp.int32)]).reshape(
            NW, _NCHUNK, 1, _CHUNK)
    return jnp.concatenate([g, s], axis=2)


def kernel(vfeat, efeat, v_reg_weight, v_reg_sum, e_reg_weight, e_reg_sum,
           in_src, in_dst, W1, b1, Wve, bve, Wev, bev,
           first_layer=1, last_layer=1):
    N, D_IN = vfeat.shape
    M = e_reg_sum.shape[0]
    E = in_src.shape[0]

    MP = 5120
    NP = 10240
    src_i = in_src.astype(jnp.int32)
    dst_i = in_dst.astype(jnp.int32)
    idx_e = _pack_idx(src_i, dst_i, MP - 1)
    idx_v = _pack_idx(dst_i, src_i, NP - 1)

    BA = 2000
    u = pl.pallas_call(
        _mlp_u_body,
        grid=(N // BA,),
        in_specs=[
            pl.BlockSpec((BA, D_IN), lambda i: (i, 0)),
            pl.BlockSpec((D_IN, D), lambda i: (0, 0)),
            pl.BlockSpec((1, D), lambda i: (0, 0)),
            pl.BlockSpec((D, D), lambda i: (0, 0)),
            pl.BlockSpec((1, D), lambda i: (0, 0)),
            pl.BlockSpec((BA, 1), lambda i: (i, 0)),
        ],
        out_specs=pl.BlockSpec((BA, D), lambda i: (i, 0)),
        out_shape=jax.ShapeDtypeStruct((N, D), jnp.float32),
    )(vfeat, W1, b1.reshape(1, D), Wve, bve.reshape(1, D), v_reg_weight)

    pe = _sc_pass(u, idx_e, MP)

    BB = 1000
    feat_e, v = pl.pallas_call(
        _edge_body,
        grid=(M // BB,),
        in_specs=[
            pl.BlockSpec((NC, BB, D), lambda i: (0, i, 0)),
            pl.BlockSpec((BB, 1), lambda i: (i, 0)),
            pl.BlockSpec((BB, 1), lambda i: (i, 0)),
            pl.BlockSpec((D, D), lambda i: (0, 0)),
            pl.BlockSpec((1, D), lambda i: (0, 0)),
        ],
        out_specs=[
            pl.BlockSpec((BB, D), lambda i: (i, 0)),
            pl.BlockSpec((BB, D), lambda i: (i, 0)),
        ],
        out_shape=[
            jax.ShapeDtypeStruct((M, D), jnp.float32),
            jax.ShapeDtypeStruct((M, D), jnp.float32),
        ],
    )(pe, e_reg_sum, e_reg_weight, Wev, bev.reshape(1, D))

    pv = _sc_pass(v, idx_v, NP)

    BC = 2000
    feat_v_out = pl.pallas_call(
        _vout_body,
        grid=(N // BC,),
        in_specs=[
            pl.BlockSpec((NC, BC, D), lambda i: (0, i, 0)),
            pl.BlockSpec((BC, 1), lambda i: (i, 0)),
        ],
        out_specs=pl.BlockSpec((BC, D), lambda i: (i, 0)),
        out_shape=jax.ShapeDtypeStruct((N, D), jnp.float32),
    )(pv, v_reg_sum)

    return (feat_v_out, feat_e)
